# baseline (speedup 1.0000x reference)
import functools

import jax
import jax.numpy as jnp
from jax import lax
from jax.experimental import pallas as pl
from jax.experimental.pallas import tpu as pltpu
from jax.experimental.pallas import tpu_sc as plsc

E = 64
D = 1024
FFN = 512
SFFN = 1024
S = 2048
P = 2 * S
TILE = 80
G_MAX = 115
X_ROWS = G_MAX * TILE
BLK = 256

NC, NS = 2, 16
NW = NC * NS
CH = P // NW
HALF = CH // 2


def _router_body(hs_ref, gw_ref, logits_ref, wcol_ref, pos_ref, texp_ref,
                 xi_ref, oh_ref):
    hs = hs_ref[...]
    gw = gw_ref[...]
    logits = lax.dot_general(hs, gw, (((1,), (1,)), ((), ())),
                             preferred_element_type=jnp.float32)
    logits_ref[...] = logits

    m = jnp.max(logits, axis=1, keepdims=True)
    ex = jnp.exp(logits - m)
    probs = ex / jnp.sum(ex, axis=1, keepdims=True)

    lane = lax.broadcasted_iota(jnp.int32, (S, E), 1).astype(jnp.float32)
    m1 = jnp.max(probs, axis=1, keepdims=True)
    i1 = jnp.min(jnp.where(probs == m1, lane, float(E)), axis=1, keepdims=True)
    probs2 = jnp.where(lane == i1, -1.0, probs)
    m2 = jnp.max(probs2, axis=1, keepdims=True)
    i2 = jnp.min(jnp.where(probs2 == m2, lane, float(E)), axis=1, keepdims=True)
    wcol_ref[0] = m1
    wcol_ref[1] = m2

    W = 128
    ep = jnp.concatenate([i1, i2], axis=0)
    erow = lax.broadcasted_iota(jnp.int32, (P, W), 1).astype(jnp.float32)
    oh_ref[...] = (erow == ep).astype(jnp.float32)
    oh = oh_ref[...]

    counts = jnp.sum(oh, axis=0, keepdims=True)
    nt = jnp.floor((counts + float(TILE - 1)) / float(TILE))
    rw = lax.broadcasted_iota(jnp.int32, (W, W), 0).astype(jnp.float32)
    cw = lax.broadcasted_iota(jnp.int32, (W, W), 1).astype(jnp.float32)
    ut = (rw <= cw).astype(jnp.float32)
    end = lax.dot_general(nt, ut, (((1,), (0,)), ((), ())),
                          preferred_element_type=jnp.float32)
    row_off = (end - nt) * float(TILE)

    r = lax.broadcasted_iota(jnp.int32, (BLK, BLK), 0).astype(jnp.float32)
    c = lax.broadcasted_iota(jnp.int32, (BLK, BLK), 1).astype(jnp.float32)
    tril = (c < r).astype(jnp.float32)
    prefix = jnp.zeros((1, W), jnp.float32)
    for i in range(P // BLK):
        blk = oh_ref[i * BLK:(i + 1) * BLK]
        intra = lax.dot_general(tril, blk, (((1,), (0,)), ((), ())),
                                preferred_element_type=jnp.float32)
        posblk = jnp.sum((intra + prefix + row_off) * blk, axis=1,
                         keepdims=True)
        pos_ref[i * BLK:(i + 1) * BLK] = posblk.astype(jnp.int32)
        prefix = prefix + jnp.sum(blk, axis=0, keepdims=True)

    total = jnp.max(end[:, :E], axis=1, keepdims=True)
    giota = lax.broadcasted_iota(jnp.int32, (G_MAX, W), 0).astype(jnp.float32)
    gclamp = jnp.minimum(giota, jnp.broadcast_to(total, (G_MAX, W)) - 1.0)
    ee = jnp.broadcast_to(end[:, :E], (G_MAX, E))
    te = jnp.sum((jnp.concatenate([ee, jnp.full((G_MAX, W - E), 1e9, jnp.float32)],
                                  axis=1) <= gclamp).astype(jnp.float32),
                 axis=1, keepdims=True)
    texp_ref[...] = jnp.minimum(te, float(E - 1)).astype(jnp.int32)
    g1 = giota[:, :1]
    xi_ref[...] = jnp.where(g1 < jnp.broadcast_to(total, (G_MAX, 1)),
                            g1, float(G_MAX)).astype(jnp.int32)


def _router(hs, gate_w):
    return pl.pallas_call(
        _router_body,
        out_shape=(
            jax.ShapeDtypeStruct((S, E), jnp.float32),
            jax.ShapeDtypeStruct((2, S, 1), jnp.float32),
            jax.ShapeDtypeStruct((P, 1), jnp.int32),
            jax.ShapeDtypeStruct((G_MAX, 1), jnp.int32),
            jax.ShapeDtypeStruct((G_MAX, 1), jnp.int32),
        ),
        scratch_shapes=[pltpu.VMEM((P, 128), jnp.float32)],
    )(hs, gate_w)


@functools.lru_cache(maxsize=None)
def _sc_kernels():
    mesh = plsc.VectorSubcoreMesh(core_axis_name="c", subcore_axis_name="s",
                                  num_cores=NC, num_subcores=NS)
    scratch = [
        pltpu.VMEM((HALF,), jnp.int32),
        pltpu.VMEM((HALF, D), jnp.float32),
        pltpu.SemaphoreType.DMA,
    ]

    @functools.partial(
        pl.kernel,
        out_type=jax.ShapeDtypeStruct((X_ROWS + TILE, D), jnp.float32),
        mesh=mesh,
        scratch_types=scratch,
    )
    def scatter_x(hs_hbm, pos_hbm, xs_hbm, idx_v, rows_v, sem):
        wid = lax.axis_index("s") * NC + lax.axis_index("c")
        base = wid * CH
        tok = lax.rem(base, S)
        for h in range(CH // HALF):
            pltpu.sync_copy(pos_hbm.at[pl.ds(base + h * HALF, HALF)], idx_v)
            pltpu.sync_copy(hs_hbm.at[pl.ds(tok + h * HALF, HALF)], rows_v)
            pltpu.async_copy(rows_v, xs_hbm.at[idx_v], sem).wait()

    @functools.partial(
        pl.kernel,
        out_type=jax.ShapeDtypeStruct((P, D), jnp.float32),
        mesh=mesh,
        scratch_types=scratch,
    )
    def gather_y(ys_hbm, pos_hbm, yp_hbm, idx_v, rows_v, sem):
        wid = lax.axis_index("s") * NC + lax.axis_index("c")
        base = wid * CH
        for h in range(CH // HALF):
            pltpu.sync_copy(pos_hbm.at[pl.ds(base + h * HALF, HALF)], idx_v)
            pltpu.async_copy(ys_hbm.at[idx_v], rows_v, sem).wait()
            pltpu.sync_copy(rows_v, yp_hbm.at[pl.ds(base + h * HALF, HALF)])

    return scatter_x, gather_y


def _expert_body(te_ref, xi_ref, x_ref, gu_ref, dw_ref, dep_ref, y_ref):
    del dep_ref
    g_id = pl.program_id(0)

    @pl.when(xi_ref[g_id] == g_id)
    def _():
        x = x_ref[...]
        gu = gu_ref[0]
        h = lax.dot_general(x, gu, (((1,), (1,)), ((), ())),
                            preferred_element_type=jnp.float32)
        g = h[:, :FFN]
        u = h[:, FFN:]
        a = g * (1.0 / (1.0 + jnp.exp(-g))) * u
        dw = dw_ref[0]
        y_ref[...] = lax.dot_general(a, dw, (((1,), (1,)), ((), ())),
                                     preferred_element_type=jnp.float32)


def _experts(texp, xi, x_sorted, gate_up_weights, down_weights, dep):
    grid_spec = pltpu.PrefetchScalarGridSpec(
        num_scalar_prefetch=2,
        grid=(G_MAX,),
        in_specs=[
            pl.BlockSpec((TILE, D), lambda g, te, xi: (xi[g], 0)),
            pl.BlockSpec((1, 2 * FFN, D), lambda g, te, xi: (te[g], 0, 0)),
            pl.BlockSpec((1, D, FFN), lambda g, te, xi: (te[g], 0, 0)),
            pl.BlockSpec((8, 128), lambda g, te, xi: (0, 0)),
        ],
        out_specs=pl.BlockSpec((TILE, D), lambda g, te, xi: (xi[g], 0)),
    )
    return pl.pallas_call(
        _expert_body,
        grid_spec=grid_spec,
        out_shape=jax.ShapeDtypeStruct((X_ROWS + TILE, D), jnp.float32),
    )(texp, xi, x_sorted, gate_up_weights, down_weights, dep)


TOKB = 256


SH = S // 2


def _shared_body(hs_ref, sgw_ref, suw_ref, sdw_ref, segw_ref, sh_ref):
    hs = hs_ref[...]
    sg = lax.dot_general(hs, sgw_ref[...], (((1,), (1,)), ((), ())),
                         preferred_element_type=jnp.float32)
    su = lax.dot_general(hs, suw_ref[...], (((1,), (1,)), ((), ())),
                         preferred_element_type=jnp.float32)
    a = sg * (1.0 / (1.0 + jnp.exp(-sg))) * su
    sh = lax.dot_general(a, sdw_ref[...], (((1,), (1,)), ((), ())),
                         preferred_element_type=jnp.float32)
    glog = jnp.sum(hs * segw_ref[...], axis=1, keepdims=True)
    gate = 1.0 / (1.0 + jnp.exp(-glog))
    sh_ref[...] = gate * sh


def _shared_half(hs, off, sgw, suw, sdw, segw, prev=None):
    nsteps = SH // TOKB
    in_specs = [
        pl.BlockSpec((TOKB, D), lambda i: (i + off, 0)),
        pl.BlockSpec((SFFN, D), lambda i: (0, 0)),
        pl.BlockSpec((SFFN, D), lambda i: (0, 0)),
        pl.BlockSpec((D, SFFN), lambda i: (0, 0)),
        pl.BlockSpec((1, D), lambda i: (0, 0)),
    ]
    args = [hs, sgw, suw, sdw, segw]
    aliases = {}
    body = _shared_body
    if prev is not None:
        def body(hs_ref, sgw_ref, suw_ref, sdw_ref, segw_ref, prev_ref,
                 sh_ref):
            del prev_ref
            _shared_body(hs_ref, sgw_ref, suw_ref, sdw_ref, segw_ref, sh_ref)

        in_specs.append(pl.BlockSpec(memory_space=pl.ANY))
        args.append(prev)
        aliases = {5: 0}
    return pl.pallas_call(
        body,
        grid=(nsteps,),
        in_specs=in_specs,
        out_specs=pl.BlockSpec((TOKB, D), lambda i: (i + off, 0)),
        out_shape=jax.ShapeDtypeStruct((S, D), jnp.float32),
        input_output_aliases=aliases,
    )(*args)


def _combine_body(sh_ref, y0_ref, y1_ref, w0_ref, w1_ref, out_ref):
    out_ref[...] = (sh_ref[...] + w0_ref[0] * y0_ref[0] + w1_ref[0] * y1_ref[0])


def _combine(sh, y3, wcol):
    return pl.pallas_call(
        _combine_body,
        grid=(S // TOKB,),
        in_specs=[
            pl.BlockSpec((TOKB, D), lambda i: (i, 0)),
            pl.BlockSpec((1, TOKB, D), lambda i: (0, i, 0)),
            pl.BlockSpec((1, TOKB, D), lambda i: (1, i, 0)),
            pl.BlockSpec((1, TOKB, 1), lambda i: (0, i, 0)),
            pl.BlockSpec((1, TOKB, 1), lambda i: (1, i, 0)),
        ],
        out_specs=pl.BlockSpec((TOKB, D), lambda i: (i, 0)),
        out_shape=jax.ShapeDtypeStruct((S, D), jnp.float32),
    )(sh, y3, y3, wcol, wcol)


def kernel(hidden_states, gate_w, gate_up_weights, down_weights,
           shared_gate_w, shared_up_w, shared_down_w, shared_expert_gate_w):
    b, s, d = hidden_states.shape
    hs = hidden_states.reshape(s, d)
    logits, wcol, pos, texp, xi = _router(hs, gate_w)
    pos_flat = pos.reshape(P)
    texp_flat = texp.reshape(G_MAX)
    xi_flat = xi.reshape(G_MAX)
    scatter_x, gather_y = _sc_kernels()
    x_sorted = scatter_x(hs, pos_flat)
    sh0 = _shared_half(hs, 0, shared_gate_w, shared_up_w, shared_down_w,
                       shared_expert_gate_w)
    y_sorted = _experts(texp_flat, xi_flat, x_sorted, gate_up_weights,
                        down_weights, sh0)
    y_pairs = gather_y(y_sorted, pos_flat)
    sh = _shared_half(hs, SH // TOKB, shared_gate_w, shared_up_w,
                      shared_down_w, shared_expert_gate_w, prev=sh0)
    y3 = y_pairs.reshape(2, S, D)
    out = _combine(sh, y3, wcol)
    return out.reshape(b, s, d), logits


# speedup vs baseline: 1.0353x; 1.0353x over previous
import functools

import jax
import jax.numpy as jnp
from jax import lax
from jax.experimental import pallas as pl
from jax.experimental.pallas import tpu as pltpu
from jax.experimental.pallas import tpu_sc as plsc

E = 64
D = 1024
FFN = 512
SFFN = 1024
S = 2048
P = 2 * S
TILE = 96
G_MAX = 107
X_ROWS = G_MAX * TILE
BLK = 256

NC, NS = 2, 16
NW = NC * NS
CH = P // NW
HALF = CH // 2


def _router_body(hs_ref, gw_ref, logits_ref, wcol_ref, pos_ref, texp_ref,
                 xi_ref, oh_ref):
    hs = hs_ref[...]
    gw = gw_ref[...]
    logits = lax.dot_general(hs, gw, (((1,), (1,)), ((), ())),
                             preferred_element_type=jnp.float32)
    logits_ref[...] = logits

    m = jnp.max(logits, axis=1, keepdims=True)
    ex = jnp.exp(logits - m)
    probs = ex / jnp.sum(ex, axis=1, keepdims=True)

    lane = lax.broadcasted_iota(jnp.int32, (S, E), 1).astype(jnp.float32)
    m1 = jnp.max(probs, axis=1, keepdims=True)
    i1 = jnp.min(jnp.where(probs == m1, lane, float(E)), axis=1, keepdims=True)
    probs2 = jnp.where(lane == i1, -1.0, probs)
    m2 = jnp.max(probs2, axis=1, keepdims=True)
    i2 = jnp.min(jnp.where(probs2 == m2, lane, float(E)), axis=1, keepdims=True)
    wcol_ref[0] = m1
    wcol_ref[1] = m2

    W = 128
    ep = jnp.concatenate([i1, i2], axis=0)
    erow = lax.broadcasted_iota(jnp.int32, (P, W), 1).astype(jnp.float32)
    oh_ref[...] = (erow == ep).astype(jnp.float32)
    oh = oh_ref[...]

    counts = jnp.sum(oh, axis=0, keepdims=True)
    nt = jnp.floor((counts + float(TILE - 1)) / float(TILE))
    rw = lax.broadcasted_iota(jnp.int32, (W, W), 0).astype(jnp.float32)
    cw = lax.broadcasted_iota(jnp.int32, (W, W), 1).astype(jnp.float32)
    ut = (rw <= cw).astype(jnp.float32)
    end = lax.dot_general(nt, ut, (((1,), (0,)), ((), ())),
                          preferred_element_type=jnp.float32)
    row_off = (end - nt) * float(TILE)

    r = lax.broadcasted_iota(jnp.int32, (BLK, BLK), 0).astype(jnp.float32)
    c = lax.broadcasted_iota(jnp.int32, (BLK, BLK), 1).astype(jnp.float32)
    tril = (c < r).astype(jnp.float32)
    prefix = jnp.zeros((1, W), jnp.float32)
    for i in range(P // BLK):
        blk = oh_ref[i * BLK:(i + 1) * BLK]
        intra = lax.dot_general(tril, blk, (((1,), (0,)), ((), ())),
                                preferred_element_type=jnp.float32)
        posblk = jnp.sum((intra + prefix + row_off) * blk, axis=1,
                         keepdims=True)
        pos_ref[i * BLK:(i + 1) * BLK] = posblk.astype(jnp.int32)
        prefix = prefix + jnp.sum(blk, axis=0, keepdims=True)

    total = jnp.max(end[:, :E], axis=1, keepdims=True)
    giota = lax.broadcasted_iota(jnp.int32, (G_MAX, W), 0).astype(jnp.float32)
    gclamp = jnp.minimum(giota, jnp.broadcast_to(total, (G_MAX, W)) - 1.0)
    ee = jnp.broadcast_to(end[:, :E], (G_MAX, E))
    te = jnp.sum((jnp.concatenate([ee, jnp.full((G_MAX, W - E), 1e9, jnp.float32)],
                                  axis=1) <= gclamp).astype(jnp.float32),
                 axis=1, keepdims=True)
    texp_ref[...] = jnp.minimum(te, float(E - 1)).astype(jnp.int32)
    g1 = giota[:, :1]
    xi_ref[...] = jnp.where(g1 < jnp.broadcast_to(total, (G_MAX, 1)),
                            g1, float(G_MAX)).astype(jnp.int32)


def _router(hs, gate_w):
    return pl.pallas_call(
        _router_body,
        out_shape=(
            jax.ShapeDtypeStruct((S, E), jnp.float32),
            jax.ShapeDtypeStruct((2, S, 1), jnp.float32),
            jax.ShapeDtypeStruct((P, 1), jnp.int32),
            jax.ShapeDtypeStruct((G_MAX, 1), jnp.int32),
            jax.ShapeDtypeStruct((G_MAX, 1), jnp.int32),
        ),
        scratch_shapes=[pltpu.VMEM((P, 128), jnp.float32)],
    )(hs, gate_w)


@functools.lru_cache(maxsize=None)
def _sc_kernels():
    mesh = plsc.VectorSubcoreMesh(core_axis_name="c", subcore_axis_name="s",
                                  num_cores=NC, num_subcores=NS)
    scratch = [
        pltpu.VMEM((HALF,), jnp.int32),
        pltpu.VMEM((HALF, D), jnp.float32),
        pltpu.SemaphoreType.DMA,
    ]

    @functools.partial(
        pl.kernel,
        out_type=jax.ShapeDtypeStruct((X_ROWS + TILE, D), jnp.float32),
        mesh=mesh,
        scratch_types=scratch,
    )
    def scatter_x(hs_hbm, pos_hbm, xs_hbm, idx_v, rows_v, sem):
        wid = lax.axis_index("s") * NC + lax.axis_index("c")
        base = wid * CH
        tok = lax.rem(base, S)
        for h in range(CH // HALF):
            pltpu.sync_copy(pos_hbm.at[pl.ds(base + h * HALF, HALF)], idx_v)
            pltpu.sync_copy(hs_hbm.at[pl.ds(tok + h * HALF, HALF)], rows_v)
            pltpu.async_copy(rows_v, xs_hbm.at[idx_v], sem).wait()

    @functools.partial(
        pl.kernel,
        out_type=jax.ShapeDtypeStruct((P, D), jnp.float32),
        mesh=mesh,
        scratch_types=scratch,
    )
    def gather_y(ys_hbm, pos_hbm, yp_hbm, idx_v, rows_v, sem):
        wid = lax.axis_index("s") * NC + lax.axis_index("c")
        base = wid * CH
        for h in range(CH // HALF):
            pltpu.sync_copy(pos_hbm.at[pl.ds(base + h * HALF, HALF)], idx_v)
            pltpu.async_copy(ys_hbm.at[idx_v], rows_v, sem).wait()
            pltpu.sync_copy(rows_v, yp_hbm.at[pl.ds(base + h * HALF, HALF)])

    return scatter_x, gather_y


def _expert_body(te_ref, xi_ref, x_ref, gu_ref, dw_ref, dep_ref, y_ref):
    del dep_ref
    g_id = pl.program_id(0)

    @pl.when(xi_ref[g_id] == g_id)
    def _():
        x = x_ref[...]
        gu = gu_ref[0]
        h = lax.dot_general(x, gu, (((1,), (1,)), ((), ())),
                            preferred_element_type=jnp.float32)
        g = h[:, :FFN]
        u = h[:, FFN:]
        a = g * (1.0 / (1.0 + jnp.exp(-g))) * u
        dw = dw_ref[0]
        y_ref[...] = lax.dot_general(a, dw, (((1,), (1,)), ((), ())),
                                     preferred_element_type=jnp.float32)


def _experts(texp, xi, x_sorted, gate_up_weights, down_weights, dep):
    grid_spec = pltpu.PrefetchScalarGridSpec(
        num_scalar_prefetch=2,
        grid=(G_MAX,),
        in_specs=[
            pl.BlockSpec((TILE, D), lambda g, te, xi: (xi[g], 0)),
            pl.BlockSpec((1, 2 * FFN, D), lambda g, te, xi: (te[g], 0, 0)),
            pl.BlockSpec((1, D, FFN), lambda g, te, xi: (te[g], 0, 0)),
            pl.BlockSpec((8, 128), lambda g, te, xi: (0, 0)),
        ],
        out_specs=pl.BlockSpec((TILE, D), lambda g, te, xi: (xi[g], 0)),
    )
    return pl.pallas_call(
        _expert_body,
        grid_spec=grid_spec,
        out_shape=jax.ShapeDtypeStruct((X_ROWS + TILE, D), jnp.float32),
    )(texp, xi, x_sorted, gate_up_weights, down_weights, dep)


TOKB = 256


SH = S // 2


def _shared_body(hs_ref, sgw_ref, suw_ref, sdw_ref, segw_ref, sh_ref):
    hs = hs_ref[...]
    sg = lax.dot_general(hs, sgw_ref[...], (((1,), (1,)), ((), ())),
                         preferred_element_type=jnp.float32)
    su = lax.dot_general(hs, suw_ref[...], (((1,), (1,)), ((), ())),
                         preferred_element_type=jnp.float32)
    a = sg * (1.0 / (1.0 + jnp.exp(-sg))) * su
    sh = lax.dot_general(a, sdw_ref[...], (((1,), (1,)), ((), ())),
                         preferred_element_type=jnp.float32)
    glog = jnp.sum(hs * segw_ref[...], axis=1, keepdims=True)
    gate = 1.0 / (1.0 + jnp.exp(-glog))
    sh_ref[...] = gate * sh


def _shared_half(hs, off, sgw, suw, sdw, segw, prev=None):
    nsteps = SH // TOKB
    in_specs = [
        pl.BlockSpec((TOKB, D), lambda i: (i + off, 0)),
        pl.BlockSpec((SFFN, D), lambda i: (0, 0)),
        pl.BlockSpec((SFFN, D), lambda i: (0, 0)),
        pl.BlockSpec((D, SFFN), lambda i: (0, 0)),
        pl.BlockSpec((1, D), lambda i: (0, 0)),
    ]
    args = [hs, sgw, suw, sdw, segw]
    aliases = {}
    body = _shared_body
    if prev is not None:
        def body(hs_ref, sgw_ref, suw_ref, sdw_ref, segw_ref, prev_ref,
                 sh_ref):
            del prev_ref
            _shared_body(hs_ref, sgw_ref, suw_ref, sdw_ref, segw_ref, sh_ref)

        in_specs.append(pl.BlockSpec(memory_space=pl.ANY))
        args.append(prev)
        aliases = {5: 0}
    return pl.pallas_call(
        body,
        grid=(nsteps,),
        in_specs=in_specs,
        out_specs=pl.BlockSpec((TOKB, D), lambda i: (i + off, 0)),
        out_shape=jax.ShapeDtypeStruct((S, D), jnp.float32),
        input_output_aliases=aliases,
    )(*args)


def _combine_body(sh_ref, y0_ref, y1_ref, w0_ref, w1_ref, out_ref):
    out_ref[...] = (sh_ref[...] + w0_ref[0] * y0_ref[0] + w1_ref[0] * y1_ref[0])


def _combine(sh, y3, wcol):
    return pl.pallas_call(
        _combine_body,
        grid=(S // TOKB,),
        in_specs=[
            pl.BlockSpec((TOKB, D), lambda i: (i, 0)),
            pl.BlockSpec((1, TOKB, D), lambda i: (0, i, 0)),
            pl.BlockSpec((1, TOKB, D), lambda i: (1, i, 0)),
            pl.BlockSpec((1, TOKB, 1), lambda i: (0, i, 0)),
            pl.BlockSpec((1, TOKB, 1), lambda i: (1, i, 0)),
        ],
        out_specs=pl.BlockSpec((TOKB, D), lambda i: (i, 0)),
        out_shape=jax.ShapeDtypeStruct((S, D), jnp.float32),
    )(sh, y3, y3, wcol, wcol)


def kernel(hidden_states, gate_w, gate_up_weights, down_weights,
           shared_gate_w, shared_up_w, shared_down_w, shared_expert_gate_w):
    b, s, d = hidden_states.shape
    hs = hidden_states.reshape(s, d)
    logits, wcol, pos, texp, xi = _router(hs, gate_w)
    pos_flat = pos.reshape(P)
    texp_flat = texp.reshape(G_MAX)
    xi_flat = xi.reshape(G_MAX)
    scatter_x, gather_y = _sc_kernels()
    x_sorted = scatter_x(hs, pos_flat)
    sh0 = _shared_half(hs, 0, shared_gate_w, shared_up_w, shared_down_w,
                       shared_expert_gate_w)
    y_sorted = _experts(texp_flat, xi_flat, x_sorted, gate_up_weights,
                        down_weights, sh0)
    y_pairs = gather_y(y_sorted, pos_flat)
    sh = _shared_half(hs, SH // TOKB, shared_gate_w, shared_up_w,
                      shared_down_w, shared_expert_gate_w, prev=sh0)
    y3 = y_pairs.reshape(2, S, D)
    out = _combine(sh, y3, wcol)
    return out.reshape(b, s, d), logits
